# baseline (speedup 1.0000x reference)
import functools

import jax
import jax.numpy as jnp
from jax import lax
from jax.experimental import pallas as pl
from jax.experimental.pallas import tpu as pltpu
from jax.experimental.pallas import tpu_sc as plsc

C1 = 17400
C2 = 100
C = C1 + C2
OUT = 40000
OUTP = 40960
T = 100
TPAD = 128
W = 64
NCHUNK = TPAD // W
NT = 16
E = 128
BPC = 4
IDXC = BPC * E
NE_RAW = 500000 + 160000
NQ = 21
NCHK = NQ * 4
EPT = NCHK * IDXC
NE = NT * EPT
ROWS_PER_TILE = OUTP // NT
OBUF_ROWS = 256


def _compute(wv, gath, contrib, woff):
    def group(g, _):
        w16 = wv[woff, pl.ds(g * 16, 16)]
        for j in range(16):
            e = g * 16 + j
            wbc = jnp.full((16,), w16[j], jnp.float32)
            wbf = plsc.pack(wbc, wbc, format=plsc.PackFormat.INTERLEAVED)
            contrib[e, pl.ds(0, 32)] = wbf * gath[e, pl.ds(0, 32)]
            contrib[e, pl.ds(32, 32)] = wbf * gath[e, pl.ds(32, 32)]
        return 0
    lax.fori_loop(0, E // 16, group, 0)


def _body(tab, rows2, cols2, ws2, zblk, out, *sc):
    colv = sc[0:4]
    rowv = sc[4:8]
    wv = sc[8:12]
    gath = sc[12:16]
    contrib = sc[16:20]
    obuf = sc[20]
    accum = sc[21]
    isem = sc[22:26]
    gsem = sc[26:30]
    ssem = sc[30:34]

    c = lax.axis_index("c")
    s = lax.axis_index("s")
    r0 = s * ROWS_PER_TILE
    choff = c * C
    crow0 = s * (EPT // E)

    pltpu.sync_copy(zblk, accum.at[pl.ds(r0, ROWS_PER_TILE)])
    plsc.subcore_barrier()

    def load_idx(cidx, buf, sync):
        rb = crow0 + cidx * BPC
        if sync:
            pltpu.sync_copy(cols2.at[pl.ds(rb, BPC)], colv[buf])
            pltpu.sync_copy(rows2.at[pl.ds(rb, BPC)], rowv[buf])
            pltpu.sync_copy(ws2.at[pl.ds(rb, BPC)], wv[buf])
        else:
            pltpu.async_copy(cols2.at[pl.ds(rb, BPC)], colv[buf], isem[buf])
            pltpu.async_copy(rows2.at[pl.ds(rb, BPC)], rowv[buf], isem[buf])
            pltpu.async_copy(ws2.at[pl.ds(rb, BPC)], wv[buf], isem[buf])

    def wait_idx(buf):
        pltpu.make_async_copy(cols2.at[pl.ds(0, BPC)], colv[buf], isem[buf]).wait()
        pltpu.make_async_copy(rows2.at[pl.ds(0, BPC)], rowv[buf], isem[buf]).wait()
        pltpu.make_async_copy(ws2.at[pl.ds(0, BPC)], wv[buf], isem[buf]).wait()

    def adjust_cols(buf):
        for r in range(BPC):
            def addoff_r(i, _, r=r):
                colv[buf][r, pl.ds(i * 16, 16)] = \
                    colv[buf][r, pl.ds(i * 16, 16)] + choff
                return 0
            lax.fori_loop(0, E // 16, addoff_r, 0)

    def issue_gather(buf, b, gb):
        pltpu.async_copy(tab.at[colv[buf].at[b]], gath[gb], gsem[gb])

    def wait_gather(gb):
        pltpu.make_async_copy(tab.at[colv[0].at[0]], gath[gb],
                              gsem[gb]).wait()

    def issue_scatter(buf, b, sb):
        pltpu.async_copy(contrib[sb], accum.at[rowv[buf].at[b]],
                         ssem[sb], add=True)

    def wait_scatter(sb):
        pltpu.make_async_copy(contrib[sb], accum.at[rowv[0].at[0]],
                              ssem[sb]).wait()

    load_idx(0, 0, sync=True)
    adjust_cols(0)
    load_idx(1, 1, sync=False)
    for b in range(BPC):
        issue_gather(0, b, b)

    def quad(kq, _):
        for qq in range(4):
            cidx = 4 * kq + qq
            nbuf = (qq + 1) % 4
            for b in range(BPC):
                if b == 0:
                    if qq == 3:
                        @pl.when(kq < NQ - 1)
                        def _():
                            wait_idx(nbuf)
                            adjust_cols(nbuf)
                    else:
                        wait_idx(nbuf)
                        adjust_cols(nbuf)
                    if qq >= 2:
                        @pl.when(kq < NQ - 1)
                        def _():
                            load_idx(cidx + 2, (qq + 2) % 4, sync=False)
                    else:
                        load_idx(cidx + 2, (qq + 2) % 4, sync=False)
                wait_gather(b)
                if qq == 0:
                    @pl.when(kq > 0)
                    def _():
                        wait_scatter(b)
                else:
                    wait_scatter(b)
                _compute(wv[qq], gath[b], contrib[b], b)
                issue_scatter(qq, b, b)
                if qq == 3:
                    @pl.when(kq < NQ - 1)
                    def _():
                        issue_gather(nbuf, b, b)
                else:
                    issue_gather(nbuf, b, b)
        return 0

    lax.fori_loop(0, NQ, quad, 0)

    for b in range(BPC):
        wait_scatter(b)
    plsc.subcore_barrier()

    def wout(k, _):
        rb = r0 + k * OBUF_ROWS
        pltpu.sync_copy(accum.at[pl.ds(rb, OBUF_ROWS)], obuf)
        pltpu.sync_copy(obuf, out.at[pl.ds(c * OUTP + rb, OBUF_ROWS)])
        return 0
    lax.fori_loop(0, ROWS_PER_TILE // OBUF_ROWS, wout, 0)


@functools.partial(
    pl.kernel,
    out_type=jax.ShapeDtypeStruct((NCHUNK * OUTP, W), jnp.bfloat16),
    mesh=plsc.VectorSubcoreMesh(core_axis_name="c", subcore_axis_name="s"),
    scratch_types=(
        [pltpu.VMEM((BPC, E), jnp.int32) for _ in range(4)]
        + [pltpu.VMEM((BPC, E), jnp.int32) for _ in range(4)]
        + [pltpu.VMEM((BPC, E), jnp.float32) for _ in range(4)]
        + [pltpu.VMEM((E, W), jnp.bfloat16) for _ in range(4)]
        + [pltpu.VMEM((E, W), jnp.bfloat16) for _ in range(4)]
        + [
            pltpu.VMEM((OBUF_ROWS, W), jnp.bfloat16),
            pltpu.VMEM_SHARED((OUTP, W), jnp.bfloat16),
        ]
        + [pltpu.SemaphoreType.DMA for _ in range(12)]
    ),
    compiler_params=pltpu.CompilerParams(
        use_tc_tiling_on_sc=False, needs_layout_passes=False),
)
def _sc_kernel(tab, rows2, cols2, ws2, zblk, out, *scratch):
    _body(tab, rows2, cols2, ws2, zblk, out, *scratch)


def kernel(lgn_spikes, bkg_spikes, lgn_rows, lgn_cols, lgn_weights,
           bkg_rows, bkg_cols, bkg_weights):
    spikes = jnp.concatenate(
        [lgn_spikes.reshape(T, C1), bkg_spikes.reshape(T, C2)], axis=1)
    spikes = jnp.pad(spikes.astype(jnp.bfloat16), ((0, TPAD - T), (0, 0)))
    tab = spikes.reshape(NCHUNK, W, C).transpose(0, 2, 1)
    tab = tab.reshape(NCHUNK * C, W)

    rows = jnp.concatenate([lgn_rows, bkg_rows])
    cols = jnp.concatenate([lgn_cols, bkg_cols + C1])
    ws = jnp.concatenate([lgn_weights, bkg_weights])
    pad = NE - NE_RAW
    rows2 = jnp.pad(rows, (0, pad)).reshape(NE // E, E)
    cols2 = jnp.pad(cols, (0, pad)).reshape(NE // E, E)
    ws2 = jnp.pad(ws, (0, pad)).reshape(NE // E, E)

    zblk = jnp.zeros((ROWS_PER_TILE, W), jnp.bfloat16)

    out = _sc_kernel(tab, rows2, cols2, ws2, zblk)

    cur = out.reshape(NCHUNK, OUTP, W).transpose(0, 2, 1).reshape(TPAD, OUTP)
    return cur[:T, :OUT].astype(jnp.float32)[None]


# speedup vs baseline: 1.5584x; 1.5584x over previous
import functools

import jax
import jax.numpy as jnp
from jax import lax
from jax.experimental import pallas as pl
from jax.experimental.pallas import tpu as pltpu
from jax.experimental.pallas import tpu_sc as plsc

C1 = 17400
C2 = 100
C = C1 + C2
OUT = 40000
OUTP = 40960
T = 100
TPAD = 128
W = 64
NCHUNK = TPAD // W
NT = 16
E = 128
BPC = 4
IDXC = BPC * E
NE_RAW = 500000 + 160000
NC3 = 27
NCHK = NC3 * 3
EPT = NCHK * IDXC
NE = NT * EPT
ROWS_PER_TILE = OUTP // NT
OBUF_ROWS = 320


def _compute(wv, gath, contrib, woff):
    def group(g, _):
        w16 = wv[woff, pl.ds(g * 16, 16)]
        for j in range(16):
            e = g * 16 + j
            wbc = jnp.full((16,), w16[j], jnp.float32)
            wbf = plsc.pack(wbc, wbc, format=plsc.PackFormat.INTERLEAVED)
            contrib[e, pl.ds(0, 32)] = wbf * gath[e, pl.ds(0, 32)]
            contrib[e, pl.ds(32, 32)] = wbf * gath[e, pl.ds(32, 32)]
        return 0
    lax.fori_loop(0, E // 16, group, 0)


def _body(tab, rows2, cols2, ws2, zblk, out, *sc):
    (colv0, colv1, colv2, rowv0, rowv1, rowv2, wv0, wv1, wv2,
     gathA, gathB, contribA, contribB, obuf, accum,
     isem0, isem1, isem2, gsemA, gsemB, ssemA, ssemB) = sc
    colv = (colv0, colv1, colv2)
    rowv = (rowv0, rowv1, rowv2)
    wv = (wv0, wv1, wv2)
    isem = (isem0, isem1, isem2)
    gath = (gathA, gathB)
    contrib = (contribA, contribB)
    gsem = (gsemA, gsemB)
    ssem = (ssemA, ssemB)

    c = lax.axis_index("c")
    s = lax.axis_index("s")
    r0 = s * ROWS_PER_TILE
    choff = c * C
    crow0 = s * (EPT // E)

    pltpu.sync_copy(zblk, accum.at[pl.ds(r0, ROWS_PER_TILE)])
    plsc.subcore_barrier()

    def load_idx(cidx, buf, sync):
        rb = crow0 + cidx * BPC
        if sync:
            pltpu.sync_copy(cols2.at[pl.ds(rb, BPC)], colv[buf])
            pltpu.sync_copy(rows2.at[pl.ds(rb, BPC)], rowv[buf])
            pltpu.sync_copy(ws2.at[pl.ds(rb, BPC)], wv[buf])
        else:
            pltpu.async_copy(cols2.at[pl.ds(rb, BPC)], colv[buf], isem[buf])
            pltpu.async_copy(rows2.at[pl.ds(rb, BPC)], rowv[buf], isem[buf])
            pltpu.async_copy(ws2.at[pl.ds(rb, BPC)], wv[buf], isem[buf])

    def wait_idx(buf):
        pltpu.make_async_copy(cols2.at[pl.ds(0, BPC)], colv[buf], isem[buf]).wait()
        pltpu.make_async_copy(rows2.at[pl.ds(0, BPC)], rowv[buf], isem[buf]).wait()
        pltpu.make_async_copy(ws2.at[pl.ds(0, BPC)], wv[buf], isem[buf]).wait()

    def adjust_cols(buf):
        for r in range(BPC):
            def addoff_r(i, _, r=r):
                colv[buf][r, pl.ds(i * 16, 16)] = \
                    colv[buf][r, pl.ds(i * 16, 16)] + choff
                return 0
            lax.fori_loop(0, E // 16, addoff_r, 0)

    def issue_gather(buf, b, gpar):
        pltpu.async_copy(tab.at[colv[buf].at[b]], gath[gpar], gsem[gpar])

    def wait_gather(gpar):
        pltpu.make_async_copy(tab.at[colv[0].at[0]], gath[gpar],
                              gsem[gpar]).wait()

    def issue_scatter(buf, b, spar):
        pltpu.async_copy(contrib[spar], accum.at[rowv[buf].at[b]],
                         ssem[spar], add=True)

    def wait_scatter(spar):
        pltpu.make_async_copy(contrib[spar], accum.at[rowv[0].at[0]],
                              ssem[spar]).wait()

    load_idx(0, 0, sync=True)
    adjust_cols(0)
    load_idx(1, 1, sync=False)
    issue_gather(0, 0, 0)

    def triple(k3, _):
        for q in range(3):
            cidx = 3 * k3 + q
            for b in range(BPC):
                gpar = b % 2
                if b == 2:
                    if q == 0:
                        load_idx(cidx + 2, (q + 2) % 3, sync=False)
                    else:
                        @pl.when(k3 < NC3 - 1)
                        def _():
                            load_idx(cidx + 2, (q + 2) % 3, sync=False)
                if b < BPC - 1:
                    issue_gather(q, b + 1, 1 - gpar)
                    wait_gather(gpar)
                else:
                    wait_gather(gpar)
                    nq = (q + 1) % 3
                    if q == 2:
                        @pl.when(k3 < NC3 - 1)
                        def _():
                            wait_idx(nq)
                            adjust_cols(nq)
                            issue_gather(nq, 0, 1 - gpar)
                    else:
                        wait_idx(nq)
                        adjust_cols(nq)
                        issue_gather(nq, 0, 1 - gpar)
                if q == 0 and b < 2:
                    @pl.when(k3 > 0)
                    def _():
                        wait_scatter(gpar)
                else:
                    wait_scatter(gpar)
                _compute(wv[q], gath[gpar], contrib[gpar], b)
                issue_scatter(q, b, gpar)
        return 0

    lax.fori_loop(0, NC3, triple, 0)

    wait_scatter(0)
    wait_scatter(1)
    plsc.subcore_barrier()

    def wout(k, _):
        rb = r0 + k * OBUF_ROWS
        pltpu.sync_copy(accum.at[pl.ds(rb, OBUF_ROWS)], obuf)
        pltpu.sync_copy(obuf, out.at[pl.ds(c * OUTP + rb, OBUF_ROWS)])
        return 0
    lax.fori_loop(0, ROWS_PER_TILE // OBUF_ROWS, wout, 0)


@functools.partial(
    pl.kernel,
    out_type=jax.ShapeDtypeStruct((NCHUNK * OUTP, W), jnp.bfloat16),
    mesh=plsc.VectorSubcoreMesh(core_axis_name="c", subcore_axis_name="s"),
    scratch_types=(
        [pltpu.VMEM((BPC, E), jnp.int32) for _ in range(3)]
        + [pltpu.VMEM((BPC, E), jnp.int32) for _ in range(3)]
        + [pltpu.VMEM((BPC, E), jnp.float32) for _ in range(3)]
        + [pltpu.VMEM((E, W), jnp.bfloat16) for _ in range(2)]
        + [pltpu.VMEM((E, W), jnp.bfloat16) for _ in range(2)]
        + [
            pltpu.VMEM((OBUF_ROWS, W), jnp.bfloat16),
            pltpu.VMEM_SHARED((OUTP, W), jnp.bfloat16),
        ]
        + [pltpu.SemaphoreType.DMA for _ in range(7)]
    ),
    compiler_params=pltpu.CompilerParams(
        use_tc_tiling_on_sc=False, needs_layout_passes=False),
)
def _sc_kernel(tab, rows2, cols2, ws2, zblk, out, *scratch):
    _body(tab, rows2, cols2, ws2, zblk, out, *scratch)


def kernel(lgn_spikes, bkg_spikes, lgn_rows, lgn_cols, lgn_weights,
           bkg_rows, bkg_cols, bkg_weights):
    spikes = jnp.concatenate(
        [lgn_spikes.reshape(T, C1), bkg_spikes.reshape(T, C2)], axis=1)
    spikes = jnp.pad(spikes.astype(jnp.bfloat16), ((0, TPAD - T), (0, 0)))
    tab = spikes.reshape(NCHUNK, W, C).transpose(0, 2, 1)
    tab = tab.reshape(NCHUNK * C, W)

    rows = jnp.concatenate([lgn_rows, bkg_rows])
    cols = jnp.concatenate([lgn_cols, bkg_cols + C1])
    ws = jnp.concatenate([lgn_weights, bkg_weights])
    pad = NE - NE_RAW
    rows2 = jnp.pad(rows, (0, pad)).reshape(NE // E, E)
    cols2 = jnp.pad(cols, (0, pad)).reshape(NE // E, E)
    ws2 = jnp.pad(ws, (0, pad)).reshape(NE // E, E)

    zblk = jnp.zeros((ROWS_PER_TILE, W), jnp.bfloat16)

    out = _sc_kernel(tab, rows2, cols2, ws2, zblk)

    cur = out.reshape(NCHUNK, OUTP, W).transpose(0, 2, 1).reshape(TPAD, OUTP)
    return cur[:T, :OUT].astype(jnp.float32)[None]


# speedup vs baseline: 1.5784x; 1.0129x over previous
import functools

import jax
import jax.numpy as jnp
from jax import lax
from jax.experimental import pallas as pl
from jax.experimental.pallas import tpu as pltpu
from jax.experimental.pallas import tpu_sc as plsc

C1 = 17400
C2 = 100
C = C1 + C2
OUT = 40000
OUTP = 40960
T = 100
TPAD = 128
W = 64
NCHUNK = TPAD // W
NT = 16
E = 128
BPC = 4
IDXC = BPC * E
NE_RAW = 500000 + 160000
NC3 = 27
NCHK = NC3 * 3
EPT = NCHK * IDXC
NE = NT * EPT
ROWS_PER_TILE = OUTP // NT
OBUF_ROWS = 320


def _compute(wv, gath, contrib, woff):
    def group(g, _):
        w16 = wv[woff, pl.ds(g * 16, 16)]
        for j in range(16):
            e = g * 16 + j
            wbc = jnp.full((16,), w16[j], jnp.float32)
            wbf = plsc.pack(wbc, wbc, format=plsc.PackFormat.INTERLEAVED)
            contrib[e, pl.ds(0, 32)] = wbf * gath[e, pl.ds(0, 32)]
            contrib[e, pl.ds(32, 32)] = wbf * gath[e, pl.ds(32, 32)]
        return 0
    lax.fori_loop(0, E // 16, group, 0)


def _body(tab, rows2, cols2, ws2, zblk, out, *sc):
    (colv0, colv1, colv2, rowv0, rowv1, rowv2, wv0, wv1, wv2,
     gathA, gathB, contribA, contribB, obuf, accum,
     isem0, isem1, isem2, gsemA, gsemB, ssemA, ssemB) = sc
    colv = (colv0, colv1, colv2)
    rowv = (rowv0, rowv1, rowv2)
    wv = (wv0, wv1, wv2)
    isem = (isem0, isem1, isem2)
    gath = (gathA, gathB)
    contrib = (contribA, contribB)
    gsem = (gsemA, gsemB)
    ssem = (ssemA, ssemB)

    c = lax.axis_index("c")
    s = lax.axis_index("s")
    r0 = s * ROWS_PER_TILE
    choff = c * C
    crow0 = s * (EPT // E)

    pltpu.sync_copy(zblk, accum.at[pl.ds(r0, ROWS_PER_TILE)])
    plsc.subcore_barrier()

    def load_idx(cidx, buf, sync):
        rb = crow0 + cidx * BPC
        if sync:
            pltpu.sync_copy(cols2.at[pl.ds(rb, BPC)], colv[buf])
            pltpu.sync_copy(rows2.at[pl.ds(rb, BPC)], rowv[buf])
            pltpu.sync_copy(ws2.at[pl.ds(rb, BPC)], wv[buf])
        else:
            pltpu.async_copy(cols2.at[pl.ds(rb, BPC)], colv[buf], isem[buf])
            pltpu.async_copy(rows2.at[pl.ds(rb, BPC)], rowv[buf], isem[buf])
            pltpu.async_copy(ws2.at[pl.ds(rb, BPC)], wv[buf], isem[buf])

    def wait_idx(buf):
        pltpu.make_async_copy(cols2.at[pl.ds(0, BPC)], colv[buf], isem[buf]).wait()
        pltpu.make_async_copy(rows2.at[pl.ds(0, BPC)], rowv[buf], isem[buf]).wait()
        pltpu.make_async_copy(ws2.at[pl.ds(0, BPC)], wv[buf], isem[buf]).wait()

    def adjust_cols(buf):
        for r in range(BPC):
            def addoff_r(i, _, r=r):
                colv[buf][r, pl.ds(i * 16, 16)] = \
                    colv[buf][r, pl.ds(i * 16, 16)] + choff
                return 0
            lax.fori_loop(0, E // 16, addoff_r, 0)

    def issue_gather(buf, b, gpar):
        pltpu.async_copy(tab.at[colv[buf].at[b]], gath[gpar], gsem[gpar])

    def wait_gather(gpar):
        pltpu.make_async_copy(tab.at[colv[0].at[0]], gath[gpar],
                              gsem[gpar]).wait()

    def issue_scatter(buf, b, spar):
        pltpu.async_copy(contrib[spar], accum.at[rowv[buf].at[b]],
                         ssem[spar], add=True)

    def wait_scatter(spar):
        pltpu.make_async_copy(contrib[spar], accum.at[rowv[0].at[0]],
                              ssem[spar]).wait()

    load_idx(0, 0, sync=True)
    adjust_cols(0)
    load_idx(1, 1, sync=False)
    issue_gather(0, 0, 0)

    def triple(k3, _):
        for q in range(3):
            cidx = 3 * k3 + q
            for b in range(BPC):
                gpar = b % 2
                nq = (q + 1) % 3
                if b == 2:
                    if q == 0:
                        load_idx(cidx + 2, (q + 2) % 3, sync=False)
                    else:
                        @pl.when(k3 < NC3 - 1)
                        def _():
                            load_idx(cidx + 2, (q + 2) % 3, sync=False)
                    if q == 2:
                        @pl.when(k3 < NC3 - 1)
                        def _():
                            wait_idx(nq)
                            adjust_cols(nq)
                    else:
                        wait_idx(nq)
                        adjust_cols(nq)
                if b < BPC - 1:
                    issue_gather(q, b + 1, 1 - gpar)
                elif q == 2:
                    @pl.when(k3 < NC3 - 1)
                    def _():
                        issue_gather(nq, 0, 1 - gpar)
                else:
                    issue_gather(nq, 0, 1 - gpar)
                wait_gather(gpar)
                if q == 0 and b < 2:
                    @pl.when(k3 > 0)
                    def _():
                        wait_scatter(gpar)
                else:
                    wait_scatter(gpar)
                _compute(wv[q], gath[gpar], contrib[gpar], b)
                issue_scatter(q, b, gpar)
        return 0

    lax.fori_loop(0, NC3, triple, 0)

    wait_scatter(0)
    wait_scatter(1)
    plsc.subcore_barrier()

    def wout(k, _):
        rb = r0 + k * OBUF_ROWS
        pltpu.sync_copy(accum.at[pl.ds(rb, OBUF_ROWS)], obuf)
        pltpu.sync_copy(obuf, out.at[pl.ds(c * OUTP + rb, OBUF_ROWS)])
        return 0
    lax.fori_loop(0, ROWS_PER_TILE // OBUF_ROWS, wout, 0)


@functools.partial(
    pl.kernel,
    out_type=jax.ShapeDtypeStruct((NCHUNK * OUTP, W), jnp.bfloat16),
    mesh=plsc.VectorSubcoreMesh(core_axis_name="c", subcore_axis_name="s"),
    scratch_types=(
        [pltpu.VMEM((BPC, E), jnp.int32) for _ in range(3)]
        + [pltpu.VMEM((BPC, E), jnp.int32) for _ in range(3)]
        + [pltpu.VMEM((BPC, E), jnp.float32) for _ in range(3)]
        + [pltpu.VMEM((E, W), jnp.bfloat16) for _ in range(2)]
        + [pltpu.VMEM((E, W), jnp.bfloat16) for _ in range(2)]
        + [
            pltpu.VMEM((OBUF_ROWS, W), jnp.bfloat16),
            pltpu.VMEM_SHARED((OUTP, W), jnp.bfloat16),
        ]
        + [pltpu.SemaphoreType.DMA for _ in range(7)]
    ),
    compiler_params=pltpu.CompilerParams(
        use_tc_tiling_on_sc=False, needs_layout_passes=False),
)
def _sc_kernel(tab, rows2, cols2, ws2, zblk, out, *scratch):
    _body(tab, rows2, cols2, ws2, zblk, out, *scratch)


def kernel(lgn_spikes, bkg_spikes, lgn_rows, lgn_cols, lgn_weights,
           bkg_rows, bkg_cols, bkg_weights):
    spikes = jnp.concatenate(
        [lgn_spikes.reshape(T, C1), bkg_spikes.reshape(T, C2)], axis=1)
    spikes = jnp.pad(spikes.astype(jnp.bfloat16), ((0, TPAD - T), (0, 0)))
    tab = spikes.reshape(NCHUNK, W, C).transpose(0, 2, 1)
    tab = tab.reshape(NCHUNK * C, W)

    rows = jnp.concatenate([lgn_rows, bkg_rows])
    cols = jnp.concatenate([lgn_cols, bkg_cols + C1])
    ws = jnp.concatenate([lgn_weights, bkg_weights])
    pad = NE - NE_RAW
    rows2 = jnp.pad(rows, (0, pad)).reshape(NE // E, E)
    cols2 = jnp.pad(cols, (0, pad)).reshape(NE // E, E)
    ws2 = jnp.pad(ws, (0, pad)).reshape(NE // E, E)

    zblk = jnp.zeros((ROWS_PER_TILE, W), jnp.bfloat16)

    out = _sc_kernel(tab, rows2, cols2, ws2, zblk)

    cur = out.reshape(NCHUNK, OUTP, W).transpose(0, 2, 1).reshape(TPAD, OUTP)
    return cur[:T, :OUT].astype(jnp.float32)[None]
